# baseline (speedup 1.0000x reference)
import functools

import jax
import jax.numpy as jnp
from jax import lax
from jax.experimental import pallas as pl
from jax.experimental.pallas import tpu as pltpu
from jax.experimental.pallas import tpu_sc as plsc

B = 16384
NNZ = 26
K = 32
NC = 2
NS = 16
D = 1000000
WS = 512
NSLAB = -(-D // WS)
DPAD = NSLAB * WS
SLAB_ITERS = -(-NSLAB // NS)
CB = 64
ROWS_PER_W = B // NS
CHUNKS = ROWS_PER_W // CB
G = CB * NNZ // 128


def _tower_body(u_hbm, v_hbm, wut_hbm, wvt_hbm, p_hbm, q_hbm,
                scru_hbm, scrv_hbm, tin_v, tout_v, idx_v, rows_v, out_v,
                sem_a, sem_b):
    cid = lax.axis_index("c")
    sid = lax.axis_index("s")

    col_k = [jnp.full((16,), k, jnp.int32) for k in range(K)]

    def transpose_table(wt_hbm, scr_hbm):
        def slab_body(i, _):
            slab = sid + NS * i

            @pl.when(slab < NSLAB)
            def _():
                c0 = slab * WS
                cps = [
                    pltpu.async_copy(wt_hbm.at[k, pl.ds(c0, WS)],
                                     tin_v.at[k], sem_a)
                    for k in range(K)
                ]
                for cp in cps:
                    cp.wait()

                def gg_body(gg, _):
                    rows16 = lax.iota(jnp.int32, 16) + gg * 16
                    for k in range(K):
                        vals = tin_v[k, pl.ds(gg * 16, 16)]
                        plsc.store_scatter(tout_v, [rows16, col_k[k]], vals)
                    return ()

                lax.fori_loop(0, WS // 16, gg_body, ())
                pltpu.sync_copy(tout_v, scr_hbm.at[pl.ds(slab * WS, WS)])

            return ()

        lax.fori_loop(0, SLAB_ITERS, slab_body, ())

    def run(idx_hbm, tab_hbm, out_hbm, base_row):
        def chunk_body(ci, _):
            row0 = base_row + ci * CB
            pltpu.sync_copy(idx_hbm.at[pl.ds(row0 * NNZ, CB * NNZ)], idx_v)
            cps = [
                pltpu.async_copy(tab_hbm.at[idx_v.at[pl.ds(g * 128, 128)]],
                                 rows_v.at[pl.ds(g * 128, 128)], sem_b)
                for g in range(G)
            ]
            for cp in cps:
                cp.wait()

            def row_body(b, _):
                i0 = b * NNZ
                acc0 = rows_v[i0, pl.ds(0, 16)]
                acc1 = rows_v[i0, pl.ds(16, 16)]
                for j in range(1, NNZ):
                    acc0 = acc0 + rows_v[i0 + j, pl.ds(0, 16)]
                    acc1 = acc1 + rows_v[i0 + j, pl.ds(16, 16)]
                out_v[b, pl.ds(0, 16)] = acc0
                out_v[b, pl.ds(16, 16)] = acc1
                return ()

            lax.fori_loop(0, CB, row_body, ())
            pltpu.sync_copy(out_v, out_hbm.at[pl.ds(row0, CB)])
            return ()

        lax.fori_loop(0, CHUNKS, chunk_body, ())

    @pl.when(cid == 0)
    def _():
        transpose_table(wut_hbm, scru_hbm)

    @pl.when(cid == 1)
    def _():
        transpose_table(wvt_hbm, scrv_hbm)

    plsc.subcore_barrier()

    @pl.when(cid == 0)
    def _():
        run(u_hbm, scru_hbm, p_hbm, sid * ROWS_PER_W)

    @pl.when(cid == 1)
    def _():
        run(v_hbm, scrv_hbm, q_hbm, sid * ROWS_PER_W)


@functools.partial(
    pl.kernel,
    out_type=(
        jax.ShapeDtypeStruct((B, K), jnp.float32),
        jax.ShapeDtypeStruct((B, K), jnp.float32),
        jax.ShapeDtypeStruct((DPAD, K), jnp.float32),
        jax.ShapeDtypeStruct((DPAD, K), jnp.float32),
    ),
    mesh=plsc.VectorSubcoreMesh(core_axis_name="c", subcore_axis_name="s",
                                num_cores=NC, num_subcores=NS),
    scratch_types=[
        pltpu.VMEM((K, WS), jnp.float32),
        pltpu.VMEM((WS, K), jnp.float32),
        pltpu.VMEM((CB * NNZ,), jnp.int32),
        pltpu.VMEM((CB * NNZ, K), jnp.float32),
        pltpu.VMEM((CB, K), jnp.float32),
        pltpu.SemaphoreType.DMA,
        pltpu.SemaphoreType.DMA,
    ],
    compiler_params=pltpu.CompilerParams(use_tc_tiling_on_sc=False,
                                         needs_layout_passes=False),
)
def _tower(u_hbm, v_hbm, wut_hbm, wvt_hbm, p_hbm, q_hbm, scru_hbm, scrv_hbm,
           tin_v, tout_v, idx_v, rows_v, out_v, sem_a, sem_b):
    _tower_body(u_hbm, v_hbm, wut_hbm, wvt_hbm, p_hbm, q_hbm,
                scru_hbm, scrv_hbm, tin_v, tout_v, idx_v, rows_v, out_v,
                sem_a, sem_b)


def kernel(U, V, Wu, Wv):
    u1 = U.astype(jnp.int32).reshape(B * NNZ)
    v1 = V.astype(jnp.int32).reshape(B * NNZ)
    wut = jnp.pad(Wu.T, ((0, 0), (0, DPAD - D)))
    wvt = jnp.pad(Wv.T, ((0, 0), (0, DPAD - D)))
    p, q, _, _ = _tower(u1, v1, wut, wvt)
    return (p, q)


# speedup vs baseline: 2.0225x; 2.0225x over previous
import functools

import jax
import jax.numpy as jnp
from jax import lax
from jax.experimental import pallas as pl
from jax.experimental.pallas import tpu as pltpu
from jax.experimental.pallas import tpu_sc as plsc

B = 16384
NNZ = 26
K = 32
NC = 2
NS = 16
NW = NC * NS
CB = 64
ROWS_PER_W = B // NS
CHUNKS = ROWS_PER_W // CB
G = CB * NNZ // 128


def _tower_body(u_hbm, v_hbm, wu_hbm, wv_hbm, p_hbm, q_hbm,
                idx_v, rows_v, out_v, sem0, sem1):
    cid = lax.axis_index("c")
    sid = lax.axis_index("s")
    sems = (sem0, sem1)

    def run(idx_hbm, tab_hbm, out_hbm, base_row):
        def fire(ci, buf):
            row0 = base_row + ci * CB
            pltpu.sync_copy(idx_hbm.at[pl.ds(row0 * NNZ, CB * NNZ)],
                            idx_v.at[buf])
            return [
                pltpu.async_copy(
                    tab_hbm.at[idx_v.at[buf, pl.ds(g * 128, 128)]],
                    rows_v.at[buf, pl.ds(g * 128, 128)], sems[buf])
                for g in range(G)
            ]

        cps = {0: fire(0, 0)}
        for ci in range(CHUNKS):
            buf = ci % 2
            if ci + 1 < CHUNKS:
                cps[(ci + 1) % 2] = fire(ci + 1, (ci + 1) % 2)
            for cp in cps[buf]:
                cp.wait()

            rv = rows_v.at[buf]

            def row_body(b, _):
                i0 = b * NNZ
                acc0 = rv[i0, pl.ds(0, 16)]
                acc1 = rv[i0, pl.ds(16, 16)]
                for j in range(1, NNZ):
                    acc0 = acc0 + rv[i0 + j, pl.ds(0, 16)]
                    acc1 = acc1 + rv[i0 + j, pl.ds(16, 16)]
                out_v[b, pl.ds(0, 16)] = acc0
                out_v[b, pl.ds(16, 16)] = acc1
                return ()

            lax.fori_loop(0, CB, row_body, ())
            pltpu.sync_copy(out_v, out_hbm.at[pl.ds(base_row + ci * CB, CB)])

    @pl.when(cid == 0)
    def _():
        run(u_hbm, wu_hbm, p_hbm, sid * ROWS_PER_W)

    @pl.when(cid == 1)
    def _():
        run(v_hbm, wv_hbm, q_hbm, sid * ROWS_PER_W)


@functools.partial(
    pl.kernel,
    out_type=(
        jax.ShapeDtypeStruct((B, K), jnp.float32),
        jax.ShapeDtypeStruct((B, K), jnp.float32),
    ),
    mesh=plsc.VectorSubcoreMesh(core_axis_name="c", subcore_axis_name="s",
                                num_cores=NC, num_subcores=NS),
    scratch_types=[
        pltpu.VMEM((2, CB * NNZ), jnp.int32),
        pltpu.VMEM((2, CB * NNZ, K), jnp.float32),
        pltpu.VMEM((CB, K), jnp.float32),
        pltpu.SemaphoreType.DMA,
        pltpu.SemaphoreType.DMA,
    ],
    compiler_params=pltpu.CompilerParams(use_tc_tiling_on_sc=False,
                                         needs_layout_passes=False),
)
def _tower(u_hbm, v_hbm, wu_hbm, wv_hbm, p_hbm, q_hbm,
           idx_v, rows_v, out_v, sem0, sem1):
    _tower_body(u_hbm, v_hbm, wu_hbm, wv_hbm, p_hbm, q_hbm,
                idx_v, rows_v, out_v, sem0, sem1)


def kernel(U, V, Wu, Wv):
    u1 = U.astype(jnp.int32).reshape(B * NNZ)
    v1 = V.astype(jnp.int32).reshape(B * NNZ)
    p, q = _tower(u1, v1, Wu, Wv)
    return (p, q)
